# baseline (speedup 1.0000x reference)
import jax
import jax.numpy as jnp
from jax import lax
from jax.experimental import pallas as pl


def _vq_body(xt_ref, embed_ref, embed_t_ref, ind_ref, qt_ref, cdt_ref):
    ft = xt_ref[0]
    c = embed_ref[...]
    ct = embed_t_ref[...]
    ab_t = lax.dot_general(c, 2.0 * ft, (((1,), (0,)), ((), ())),
                           preferred_element_type=jnp.float32)
    f2 = jnp.sum(ft * ft, axis=0, keepdims=True)
    c2 = jnp.sum(c * c, axis=1)[:, None]
    dist_t = (f2 - ab_t) + c2
    m = jnp.min(dist_t, axis=0, keepdims=True)
    kidx = lax.broadcasted_iota(jnp.int32, dist_t.shape, 0)
    ind = jnp.min(jnp.where(dist_t <= m, kidx, dist_t.shape[0]), axis=0)
    ind_ref[...] = ind
    onehot_t = (kidx == ind[None, :]).astype(jnp.bfloat16)
    c_hi = ct.astype(jnp.bfloat16)
    r1 = ct - c_hi.astype(jnp.float32)
    c_mid = r1.astype(jnp.bfloat16)
    c_lo = (r1 - c_mid.astype(jnp.float32)).astype(jnp.bfloat16)
    dn = (((1,), (0,)), ((), ()))

    def _mm(a):
        return lax.dot_general(a, onehot_t, dn,
                               preferred_element_type=jnp.float32)

    qt = (_mm(c_hi) + _mm(c_mid)) + _mm(c_lo)
    qt_ref[...] = qt
    cdt_ref[...] = qt - ft


@jax.jit
def kernel(x, embed):
    d = x.shape[-1]
    k = embed.shape[0]
    n = x.shape[0] * x.shape[1]
    tn = x.shape[1]
    xt = jnp.transpose(x, (0, 2, 1))
    embed_t = embed.T
    ind, qt, cdt = pl.pallas_call(
        _vq_body,
        grid=(n // tn,),
        in_specs=[
            pl.BlockSpec((1, d, tn), lambda i: (i, 0, 0)),
            pl.BlockSpec((k, d), lambda i: (0, 0)),
            pl.BlockSpec((d, k), lambda i: (0, 0)),
        ],
        out_specs=[
            pl.BlockSpec((tn,), lambda i: (i,)),
            pl.BlockSpec((d, tn), lambda i: (0, i)),
            pl.BlockSpec((d, tn), lambda i: (0, i)),
        ],
        out_shape=[
            jax.ShapeDtypeStruct((n,), jnp.int32),
            jax.ShapeDtypeStruct((d, n), jnp.float32),
            jax.ShapeDtypeStruct((d, n), jnp.float32),
        ],
    )(xt, embed, embed_t)
    return (qt.T, ind, cdt.T)


# speedup vs baseline: 1.3370x; 1.3370x over previous
import jax
import jax.numpy as jnp
from jax import lax
from jax.experimental import pallas as pl

_SLABS = 2


def _vq_body(xt_ref, embed_ref, embed_t_ref, ind_ref, qt_ref, cdt_ref):
    c = embed_ref[...]
    ct = embed_t_ref[...]
    c2 = jnp.sum(c * c, axis=1)[:, None]
    tn = xt_ref.shape[2]
    for s in range(_SLABS):
        ft = xt_ref[s]
        ab_t = lax.dot_general(c, 2.0 * ft, (((1,), (0,)), ((), ())),
                               preferred_element_type=jnp.float32)
        f2 = jnp.sum(ft * ft, axis=0, keepdims=True)
        dist_t = (f2 - ab_t) + c2
        m = jnp.min(dist_t, axis=0, keepdims=True)
        kidx = lax.broadcasted_iota(jnp.int32, dist_t.shape, 0)
        ind = jnp.min(jnp.where(dist_t <= m, kidx, dist_t.shape[0]), axis=0)
        ind_ref[pl.ds(s * tn, tn)] = ind
        onehot_t = (kidx == ind[None, :]).astype(jnp.float32)
        qt = lax.dot_general(ct, onehot_t, (((1,), (0,)), ((), ())),
                             preferred_element_type=jnp.float32)
        qt_ref[:, pl.ds(s * tn, tn)] = qt
        cdt_ref[:, pl.ds(s * tn, tn)] = qt - ft


@jax.jit
def kernel(x, embed):
    d = x.shape[-1]
    k = embed.shape[0]
    n = x.shape[0] * x.shape[1]
    tn = x.shape[1]
    xt = jnp.transpose(x, (0, 2, 1))
    embed_t = embed.T
    ind, qt, cdt = pl.pallas_call(
        _vq_body,
        grid=(n // (tn * _SLABS),),
        in_specs=[
            pl.BlockSpec((_SLABS, d, tn), lambda i: (i, 0, 0)),
            pl.BlockSpec((k, d), lambda i: (0, 0)),
            pl.BlockSpec((d, k), lambda i: (0, 0)),
        ],
        out_specs=[
            pl.BlockSpec((_SLABS * tn,), lambda i: (i,)),
            pl.BlockSpec((d, _SLABS * tn), lambda i: (0, i)),
            pl.BlockSpec((d, _SLABS * tn), lambda i: (0, i)),
        ],
        out_shape=[
            jax.ShapeDtypeStruct((n,), jnp.int32),
            jax.ShapeDtypeStruct((d, n), jnp.float32),
            jax.ShapeDtypeStruct((d, n), jnp.float32),
        ],
    )(xt, embed, embed_t)
    return (qt.T, ind, cdt.T)


# speedup vs baseline: 1.3487x; 1.0087x over previous
import jax
import jax.numpy as jnp
from jax import lax
from jax.experimental import pallas as pl

_SLABS = 4


def _vq_body(xt_ref, embed_ref, embed_t_ref, ind_ref, qt_ref, cdt_ref):
    c = embed_ref[...]
    ct = embed_t_ref[...]
    c2 = jnp.sum(c * c, axis=1)[:, None]
    tn = xt_ref.shape[2]
    for s in range(_SLABS):
        ft = xt_ref[s]
        ab_t = lax.dot_general(c, 2.0 * ft, (((1,), (0,)), ((), ())),
                               preferred_element_type=jnp.float32)
        f2 = jnp.sum(ft * ft, axis=0, keepdims=True)
        dist_t = (f2 - ab_t) + c2
        m = jnp.min(dist_t, axis=0, keepdims=True)
        kidx = lax.broadcasted_iota(jnp.int32, dist_t.shape, 0)
        ind = jnp.min(jnp.where(dist_t <= m, kidx, dist_t.shape[0]), axis=0)
        ind_ref[pl.ds(s * tn, tn)] = ind
        onehot_t = (kidx == ind[None, :]).astype(jnp.float32)
        qt = lax.dot_general(ct, onehot_t, (((1,), (0,)), ((), ())),
                             preferred_element_type=jnp.float32)
        qt_ref[:, pl.ds(s * tn, tn)] = qt
        cdt_ref[:, pl.ds(s * tn, tn)] = qt - ft


@jax.jit
def kernel(x, embed):
    d = x.shape[-1]
    k = embed.shape[0]
    n = x.shape[0] * x.shape[1]
    tn = x.shape[1]
    xt = jnp.transpose(x, (0, 2, 1))
    embed_t = embed.T
    ind, qt, cdt = pl.pallas_call(
        _vq_body,
        grid=(n // (tn * _SLABS),),
        in_specs=[
            pl.BlockSpec((_SLABS, d, tn), lambda i: (i, 0, 0)),
            pl.BlockSpec((k, d), lambda i: (0, 0)),
            pl.BlockSpec((d, k), lambda i: (0, 0)),
        ],
        out_specs=[
            pl.BlockSpec((_SLABS * tn,), lambda i: (i,)),
            pl.BlockSpec((d, _SLABS * tn), lambda i: (0, i)),
            pl.BlockSpec((d, _SLABS * tn), lambda i: (0, i)),
        ],
        out_shape=[
            jax.ShapeDtypeStruct((n,), jnp.int32),
            jax.ShapeDtypeStruct((d, n), jnp.float32),
            jax.ShapeDtypeStruct((d, n), jnp.float32),
        ],
    )(xt, embed, embed_t)
    return (qt.T, ind, cdt.T)
